# Initial kernel scaffold; baseline (speedup 1.0000x reference)
import jax
import jax.numpy as jnp
from jax.experimental import pallas as pl


def kernel(state, conv1_w, conv1_b, conv2_w, conv2_b, conv3_w, conv3_b, fc1_w, fc1_b, fc2_w, fc2_b):
    raise NotImplementedError("write your pallas kernel here")


# speedup vs baseline: 1.6642x; 1.6642x over previous
import jax
import jax.numpy as jnp
from jax.experimental import pallas as pl
from jax.experimental.pallas import tpu as pltpu

_H, _W = 8, 17
_C_IN = 13
_H1, _W1 = 5, 14
_H2, _W2 = 4, 13
_BT = 48


def _convs_kernel(p1_ref, w1_ref, b1_ref, w2_ref, b2_ref, w3_ref, b3_ref,
                  o_ref):
    a1 = jnp.dot(p1_ref[...], w1_ref[...], preferred_element_type=jnp.float32)
    a1 = jnp.maximum(a1 + b1_ref[...], 0.0).astype(jnp.bfloat16)
    a1 = a1.reshape(_BT, _H1, _W1, 64)
    p2 = jnp.concatenate(
        [a1[:, i:i + _H2, j:j + _W2, :] for i in (0, 1) for j in (0, 1)],
        axis=-1).reshape(_BT * _H2 * _W2, 256)
    a2 = jnp.dot(p2, w2_ref[...], preferred_element_type=jnp.float32)
    a2 = jnp.maximum(a2 + b2_ref[...], 0.0).astype(jnp.bfloat16)
    a3 = jnp.dot(a2, w3_ref[...], preferred_element_type=jnp.float32)
    a3 = jnp.maximum(a3 + b3_ref[...], 0.0)
    o_ref[...] = a3.astype(o_ref.dtype)


def _fc_kernel(x_ref, w1_ref, b1_ref, w2_ref, b2_ref, o_ref, acc_ref):
    k = pl.program_id(1)

    @pl.when(k == 0)
    def _():
        acc_ref[...] = jnp.zeros_like(acc_ref)

    acc_ref[...] += jnp.dot(x_ref[...].astype(jnp.float32), w1_ref[...],
                            preferred_element_type=jnp.float32)

    @pl.when(k == pl.num_programs(1) - 1)
    def _():
        h = jnp.maximum(acc_ref[...] + b1_ref[...], 0.0)
        r = jnp.dot(h, w2_ref[...], preferred_element_type=jnp.float32)
        o_ref[...] = (r + b2_ref[...]).astype(o_ref.dtype)


def kernel(state, conv1_w, conv1_b, conv2_w, conv2_b, conv3_w, conv3_b,
           fc1_w, fc1_b, fc2_w, fc2_b):
    n = state.shape[0]
    x = jnp.transpose(state, (0, 2, 3, 1)).astype(jnp.bfloat16)

    parts = [x[:, i:i + _H1, j:j + _W1, :] for i in range(4) for j in range(4)]
    kpad = conv1_w.shape[0] - 16 * _C_IN
    parts.append(jnp.zeros((n, _H1, _W1, kpad), jnp.bfloat16))
    p1 = jnp.concatenate(parts, axis=-1).reshape(n * _H1 * _W1, conv1_w.shape[0])

    grid1 = n // _BT
    m1 = _BT * _H1 * _W1
    m2 = _BT * _H2 * _W2
    a3 = pl.pallas_call(
        _convs_kernel,
        out_shape=jax.ShapeDtypeStruct((grid1 * m2, 128), jnp.bfloat16),
        grid=(grid1,),
        in_specs=[
            pl.BlockSpec((m1, 256), lambda i: (i, 0)),
            pl.BlockSpec((256, 64), lambda i: (0, 0)),
            pl.BlockSpec((1, 64), lambda i: (0, 0)),
            pl.BlockSpec((256, 128), lambda i: (0, 0)),
            pl.BlockSpec((1, 128), lambda i: (0, 0)),
            pl.BlockSpec((128, 128), lambda i: (0, 0)),
            pl.BlockSpec((1, 128), lambda i: (0, 0)),
        ],
        out_specs=pl.BlockSpec((m2, 128), lambda i: (i, 0)),
        compiler_params=pltpu.CompilerParams(
            dimension_semantics=("parallel",)),
    )(p1, conv1_w.astype(jnp.bfloat16), conv1_b,
      conv2_w.astype(jnp.bfloat16), conv2_b,
      conv3_w.astype(jnp.bfloat16), conv3_b)

    flat = a3.reshape(n, _H2 * _W2 * 128)

    kdim = fc1_w.shape[0]
    tk = 1664
    mhalf = n // 2
    out = pl.pallas_call(
        _fc_kernel,
        out_shape=jax.ShapeDtypeStruct((n, fc2_w.shape[1]), jnp.float32),
        grid=(2, kdim // tk),
        in_specs=[
            pl.BlockSpec((mhalf, tk), lambda m, k: (m, k)),
            pl.BlockSpec((tk, 512), lambda m, k: (k, 0)),
            pl.BlockSpec((1, 512), lambda m, k: (0, 0)),
            pl.BlockSpec((512, fc2_w.shape[1]), lambda m, k: (0, 0)),
            pl.BlockSpec((1, fc2_w.shape[1]), lambda m, k: (0, 0)),
        ],
        out_specs=pl.BlockSpec((mhalf, fc2_w.shape[1]), lambda m, k: (m, 0)),
        scratch_shapes=[pltpu.VMEM((mhalf, 512), jnp.float32)],
        compiler_params=pltpu.CompilerParams(
            dimension_semantics=("parallel", "arbitrary")),
    )(flat, fc1_w, fc1_b, fc2_w, fc2_b)

    return out[:n, :6]


# speedup vs baseline: 4.8961x; 2.9420x over previous
import jax
import jax.numpy as jnp
from jax.experimental import pallas as pl
from jax.experimental.pallas import tpu as pltpu

_H, _W = 8, 17
_S = _H * _W
_C = 13
_BT = 48
_L = _BT * _S


def _convs_kernel(xt_ref, w1_ref, w2_ref, w3_ref, o_ref):
    xv = xt_ref[...]
    f32 = jnp.float32

    a1 = jnp.dot(w1_ref[0], xv, preferred_element_type=f32)
    for t in range(1, 16):
        off = (t // 4) * _W + (t % 4)
        a1 += jnp.dot(w1_ref[t], jnp.roll(xv, -off, axis=1),
                      preferred_element_type=f32)
    a1 = jnp.maximum(a1, 0.0).astype(jnp.bfloat16)
    a1 = jnp.concatenate([a1, jnp.ones((1, _L), jnp.bfloat16)], axis=0)

    a2 = jnp.dot(w2_ref[0], a1, preferred_element_type=f32)
    for t in range(1, 4):
        off = (t // 2) * _W + (t % 2)
        a2 += jnp.dot(w2_ref[t], jnp.roll(a1, -off, axis=1),
                      preferred_element_type=f32)
    a2 = jnp.maximum(a2, 0.0).astype(jnp.bfloat16)
    a2 = jnp.concatenate([a2, jnp.ones((1, _L), jnp.bfloat16)], axis=0)

    a3 = jnp.dot(w3_ref[...], a2, preferred_element_type=f32)
    a3 = jnp.maximum(a3, 0.0)

    rows = a3.T.reshape(_BT, _H, _W, 128)
    out = rows[:, :4, :13, :].reshape(_BT * 52, 128)
    o_ref[...] = out.astype(o_ref.dtype)


def _fc_kernel(x_ref, w1_ref, b1_ref, w2_ref, b2_ref, o_ref, acc_ref):
    k = pl.program_id(1)

    @pl.when(k == 0)
    def _():
        acc_ref[...] = jnp.zeros_like(acc_ref)

    acc_ref[...] += jnp.dot(x_ref[...].astype(jnp.float32), w1_ref[...],
                            preferred_element_type=jnp.float32)

    @pl.when(k == pl.num_programs(1) - 1)
    def _():
        h = jnp.maximum(acc_ref[...] + b1_ref[...], 0.0)
        r = jnp.dot(h, w2_ref[...], preferred_element_type=jnp.float32)
        o_ref[...] = (r + b2_ref[...]).astype(o_ref.dtype)


def kernel(state, conv1_w, conv1_b, conv2_w, conv2_b, conv3_w, conv3_b,
           fc1_w, fc1_b, fc2_w, fc2_b):
    n = state.shape[0]
    bf16 = jnp.bfloat16

    xt = state.astype(bf16).reshape(n, _C, _S).transpose(1, 0, 2)
    xt = xt.reshape(_C, n * _S)
    xt = jnp.concatenate([xt, jnp.ones((1, n * _S), bf16)], axis=0)

    w1e = jnp.zeros((16, 64, 14), jnp.float32)
    w1e = w1e.at[:, :, :_C].set(
        conv1_w[:16 * _C].reshape(16, _C, 64).transpose(0, 2, 1))
    w1e = w1e.at[0, :, _C].set(conv1_b[0]).astype(bf16)
    w2e = jnp.zeros((4, 128, 65), jnp.float32)
    w2e = w2e.at[:, :, :64].set(conv2_w.reshape(4, 64, 128).transpose(0, 2, 1))
    w2e = w2e.at[0, :, 64].set(conv2_b[0]).astype(bf16)
    w3e = jnp.concatenate([conv3_w.T, conv3_b.T], axis=1).astype(bf16)

    grid1 = n // _BT
    a3 = pl.pallas_call(
        _convs_kernel,
        out_shape=jax.ShapeDtypeStruct((grid1 * _BT * 52, 128), bf16),
        grid=(grid1,),
        in_specs=[
            pl.BlockSpec((14, _L), lambda i: (0, i)),
            pl.BlockSpec((16, 64, 14), lambda i: (0, 0, 0)),
            pl.BlockSpec((4, 128, 65), lambda i: (0, 0, 0)),
            pl.BlockSpec((128, 129), lambda i: (0, 0)),
        ],
        out_specs=pl.BlockSpec((_BT * 52, 128), lambda i: (i, 0)),
        compiler_params=pltpu.CompilerParams(
            dimension_semantics=("parallel",)),
    )(xt, w1e, w2e, w3e)

    flat = a3.reshape(n, 52 * 128)

    kdim = fc1_w.shape[0]
    tk = 1664
    mhalf = n // 2
    out = pl.pallas_call(
        _fc_kernel,
        out_shape=jax.ShapeDtypeStruct((n, fc2_w.shape[1]), jnp.float32),
        grid=(2, kdim // tk),
        in_specs=[
            pl.BlockSpec((mhalf, tk), lambda m, k: (m, k)),
            pl.BlockSpec((tk, 512), lambda m, k: (k, 0)),
            pl.BlockSpec((1, 512), lambda m, k: (0, 0)),
            pl.BlockSpec((512, fc2_w.shape[1]), lambda m, k: (0, 0)),
            pl.BlockSpec((1, fc2_w.shape[1]), lambda m, k: (0, 0)),
        ],
        out_specs=pl.BlockSpec((mhalf, fc2_w.shape[1]), lambda m, k: (m, 0)),
        scratch_shapes=[pltpu.VMEM((mhalf, 512), jnp.float32)],
        compiler_params=pltpu.CompilerParams(
            dimension_semantics=("parallel", "arbitrary")),
    )(flat, fc1_w, fc1_b, fc2_w, fc2_b)

    return out[:n, :6]


# speedup vs baseline: 8.1542x; 1.6654x over previous
import jax
import jax.numpy as jnp
from jax.experimental import pallas as pl
from jax.experimental.pallas import tpu as pltpu

_H, _W = 8, 17
_S = _H * _W
_C = 13
_BT = 48
_L = _BT * _S

_OFFS1 = [i * _W + j for i in range(4) for j in range(4)]
_OFFS2 = [i * _W + j for i in range(2) for j in range(2)]


def _dot0(w, x):
    return jax.lax.dot_general(
        w, x, (((0,), (0,)), ((), ())),
        preferred_element_type=jnp.float32)


def _convs_kernel(xt_ref, w1_ref, b1_ref, w2_ref, b2_ref, w3_ref, b3_ref,
                  o_ref):
    bf16 = jnp.bfloat16
    xv = xt_ref[...]

    x1 = jnp.concatenate(
        [jnp.roll(xv, -off, axis=1) if off else xv for off in _OFFS1], axis=0)
    a1 = _dot0(w1_ref[0:16 * _C, :].astype(bf16), x1)
    a1 = jnp.maximum(a1 + b1_ref[...].T, 0.0).astype(bf16)

    x2 = jnp.concatenate(
        [jnp.roll(a1, -off, axis=1) if off else a1 for off in _OFFS2], axis=0)
    a2 = _dot0(w2_ref[...].astype(bf16), x2)
    a2 = jnp.maximum(a2 + b2_ref[...].T, 0.0).astype(bf16)

    a3 = _dot0(w3_ref[...].astype(bf16), a2)
    a3 = jnp.maximum(a3 + b3_ref[...].T, 0.0).astype(bf16)

    rows = a3.T.reshape(_BT, _S, 128)
    keep = jnp.concatenate(
        [rows[:, h * _W:h * _W + 13, :] for h in range(4)], axis=1)
    o_ref[...] = keep.reshape(_BT * 52, 128)


def _fc_kernel(x_ref, w1_ref, b1_ref, w2_ref, b2_ref, o_ref, acc_ref):
    k = pl.program_id(1)

    @pl.when(k == 0)
    def _():
        acc_ref[...] = jnp.zeros_like(acc_ref)

    acc_ref[...] += jnp.dot(x_ref[...], w1_ref[...].astype(jnp.bfloat16),
                            preferred_element_type=jnp.float32)

    @pl.when(k == pl.num_programs(1) - 1)
    def _():
        h = jnp.maximum(acc_ref[...] + b1_ref[...], 0.0)
        r = jnp.dot(h, w2_ref[...], preferred_element_type=jnp.float32)
        o_ref[...] = (r + b2_ref[...]).astype(o_ref.dtype)


def kernel(state, conv1_w, conv1_b, conv2_w, conv2_b, conv3_w, conv3_b,
           fc1_w, fc1_b, fc2_w, fc2_b):
    n = state.shape[0]
    bf16 = jnp.bfloat16

    xt = state.astype(bf16).reshape(n, _C, _S).transpose(1, 0, 2)
    xt = xt.reshape(_C, n * _S)

    grid1 = n // _BT
    a3 = pl.pallas_call(
        _convs_kernel,
        out_shape=jax.ShapeDtypeStruct((grid1 * _BT * 52, 128), bf16),
        grid=(grid1,),
        in_specs=[
            pl.BlockSpec((_C, _L), lambda i: (0, i)),
            pl.BlockSpec(conv1_w.shape, lambda i: (0, 0)),
            pl.BlockSpec((1, 64), lambda i: (0, 0)),
            pl.BlockSpec(conv2_w.shape, lambda i: (0, 0)),
            pl.BlockSpec((1, 128), lambda i: (0, 0)),
            pl.BlockSpec(conv3_w.shape, lambda i: (0, 0)),
            pl.BlockSpec((1, 128), lambda i: (0, 0)),
        ],
        out_specs=pl.BlockSpec((_BT * 52, 128), lambda i: (i, 0)),
        compiler_params=pltpu.CompilerParams(
            dimension_semantics=("arbitrary",)),
    )(xt, conv1_w, conv1_b, conv2_w, conv2_b, conv3_w, conv3_b)

    flat = a3.reshape(n, 52 * 128)

    kdim = fc1_w.shape[0]
    tk = 1664
    mhalf = n // 2
    out = pl.pallas_call(
        _fc_kernel,
        out_shape=jax.ShapeDtypeStruct((n, fc2_w.shape[1]), jnp.float32),
        grid=(2, kdim // tk),
        in_specs=[
            pl.BlockSpec((mhalf, tk), lambda m, k: (m, k)),
            pl.BlockSpec((tk, 512), lambda m, k: (k, 0)),
            pl.BlockSpec((1, 512), lambda m, k: (0, 0)),
            pl.BlockSpec((512, fc2_w.shape[1]), lambda m, k: (0, 0)),
            pl.BlockSpec((1, fc2_w.shape[1]), lambda m, k: (0, 0)),
        ],
        out_specs=pl.BlockSpec((mhalf, fc2_w.shape[1]), lambda m, k: (m, 0)),
        scratch_shapes=[pltpu.VMEM((mhalf, 512), jnp.float32)],
        compiler_params=pltpu.CompilerParams(
            dimension_semantics=("arbitrary", "arbitrary")),
    )(flat, fc1_w, fc1_b, fc2_w, fc2_b)

    return out[:n, :6]


# speedup vs baseline: 9.9860x; 1.2247x over previous
import jax
import jax.numpy as jnp
from jax.experimental import pallas as pl
from jax.experimental.pallas import tpu as pltpu

_H, _W = 8, 17
_S = _H * _W
_C = 13
_BT = 48
_L = _BT * _S

_OFFS1 = [i * _W + j for i in range(4) for j in range(4)]
_OFFS2 = [i * _W + j for i in range(2) for j in range(2)]


def _dot0(w, x):
    return jax.lax.dot_general(
        w, x, (((0,), (0,)), ((), ())),
        preferred_element_type=jnp.float32)


def _qnet_kernel(xt_ref, w1_ref, b1_ref, w2_ref, b2_ref, w3_ref, b3_ref,
                 fw1_ref, fb1_ref, fw2_ref, fb2_ref, o_ref, fw1bf_ref):
    bf16 = jnp.bfloat16

    @pl.when(pl.program_id(0) == 0)
    def _():
        fw1bf_ref[...] = fw1_ref[...].astype(bf16)

    xv = xt_ref[...]

    x1 = jnp.concatenate(
        [jnp.roll(xv, -off, axis=1) if off else xv for off in _OFFS1], axis=0)
    a1 = _dot0(w1_ref[0:16 * _C, :].astype(bf16), x1)
    a1 = jnp.maximum(a1 + b1_ref[...].T, 0.0).astype(bf16)

    x2 = jnp.concatenate(
        [jnp.roll(a1, -off, axis=1) if off else a1 for off in _OFFS2], axis=0)
    a2 = _dot0(w2_ref[...].astype(bf16), x2)
    a2 = jnp.maximum(a2 + b2_ref[...].T, 0.0).astype(bf16)

    a3 = _dot0(w3_ref[...].astype(bf16), a2)
    a3 = jnp.maximum(a3 + b3_ref[...].T, 0.0).astype(bf16)

    rows = a3.T.reshape(_BT, _S, 128)
    keep = jnp.concatenate(
        [rows[:, h * _W:h * _W + 13, :] for h in range(4)], axis=1)

    w1b = fw1bf_ref[...]
    acc = jnp.dot(keep[:, 0, :], w1b[0:128, :],
                  preferred_element_type=jnp.float32)
    for p in range(1, 52):
        acc += jnp.dot(keep[:, p, :], w1b[128 * p:128 * (p + 1), :],
                       preferred_element_type=jnp.float32)
    h = jnp.maximum(acc + fb1_ref[...], 0.0).astype(bf16)
    r = jnp.dot(h, fw2_ref[...].astype(bf16),
                preferred_element_type=jnp.float32)
    o_ref[...] = r + fb2_ref[...]


def kernel(state, conv1_w, conv1_b, conv2_w, conv2_b, conv3_w, conv3_b,
           fc1_w, fc1_b, fc2_w, fc2_b):
    n = state.shape[0]
    bf16 = jnp.bfloat16

    xt = state.astype(bf16).reshape(n, _C, _S).transpose(1, 0, 2)
    xt = xt.reshape(_C, n * _S)

    grid1 = n // _BT
    np_ = fc2_w.shape[1]
    out = pl.pallas_call(
        _qnet_kernel,
        out_shape=jax.ShapeDtypeStruct((n, np_), jnp.float32),
        grid=(grid1,),
        in_specs=[
            pl.BlockSpec((_C, _L), lambda i: (0, i)),
            pl.BlockSpec(conv1_w.shape, lambda i: (0, 0)),
            pl.BlockSpec((1, 64), lambda i: (0, 0)),
            pl.BlockSpec(conv2_w.shape, lambda i: (0, 0)),
            pl.BlockSpec((1, 128), lambda i: (0, 0)),
            pl.BlockSpec(conv3_w.shape, lambda i: (0, 0)),
            pl.BlockSpec((1, 128), lambda i: (0, 0)),
            pl.BlockSpec(fc1_w.shape, lambda i: (0, 0)),
            pl.BlockSpec((1, 512), lambda i: (0, 0)),
            pl.BlockSpec(fc2_w.shape, lambda i: (0, 0)),
            pl.BlockSpec((1, np_), lambda i: (0, 0)),
        ],
        out_specs=pl.BlockSpec((_BT, np_), lambda i: (i, 0)),
        scratch_shapes=[pltpu.VMEM(fc1_w.shape, bf16)],
        compiler_params=pltpu.CompilerParams(
            dimension_semantics=("arbitrary",)),
    )(xt, conv1_w, conv1_b, conv2_w, conv2_b, conv3_w, conv3_b,
      fc1_w, fc1_b, fc2_w, fc2_b)

    return out[:n, :6]
